# Initial kernel scaffold; baseline (speedup 1.0000x reference)
import jax
import jax.numpy as jnp
from jax.experimental import pallas as pl


def kernel(x, position_embeddings):
    raise NotImplementedError("write your pallas kernel here")


# speedup vs baseline: 1.6709x; 1.6709x over previous
import jax
import jax.numpy as jnp
from jax.experimental import pallas as pl


def _add_body(x_ref, p_ref, o_ref):
    o_ref[...] = x_ref[...] + p_ref[...]


def kernel(x, position_embeddings):
    B, S, H = x.shape
    pos = position_embeddings[:S]
    BS = 512
    grid = (S // BS, B)
    return pl.pallas_call(
        _add_body,
        grid=grid,
        in_specs=[
            pl.BlockSpec((1, BS, H), lambda i, j: (j, i, 0)),
            pl.BlockSpec((BS, H), lambda i, j: (i, 0)),
        ],
        out_specs=pl.BlockSpec((1, BS, H), lambda i, j: (j, i, 0)),
        out_shape=jax.ShapeDtypeStruct((B, S, H), x.dtype),
    )(x, pos)


# speedup vs baseline: 1.8448x; 1.1041x over previous
import jax
import jax.numpy as jnp
from jax.experimental import pallas as pl


def _add_body(x_ref, p_ref, o_ref):
    o_ref[...] = x_ref[...] + p_ref[...]


def kernel(x, position_embeddings):
    B, S, H = x.shape
    pos = position_embeddings[:S]
    BS = 1024
    grid = (S // BS, B)
    return pl.pallas_call(
        _add_body,
        grid=grid,
        in_specs=[
            pl.BlockSpec((1, BS, H), lambda i, j: (j, i, 0)),
            pl.BlockSpec((BS, H), lambda i, j: (i, 0)),
        ],
        out_specs=pl.BlockSpec((1, BS, H), lambda i, j: (j, i, 0)),
        out_shape=jax.ShapeDtypeStruct((B, S, H), x.dtype),
    )(x, pos)


# speedup vs baseline: 1.9681x; 1.0668x over previous
import jax
import jax.numpy as jnp
from jax.experimental import pallas as pl


def _add_body(x_ref, p_ref, o_ref):
    o_ref[...] = x_ref[...] + p_ref[...]


def kernel(x, position_embeddings):
    B, S, H = x.shape
    pos = position_embeddings[:S]
    BS = 2048
    grid = (S // BS, B)
    return pl.pallas_call(
        _add_body,
        grid=grid,
        in_specs=[
            pl.BlockSpec((1, BS, H), lambda i, j: (j, i, 0)),
            pl.BlockSpec((BS, H), lambda i, j: (i, 0)),
        ],
        out_specs=pl.BlockSpec((1, BS, H), lambda i, j: (j, i, 0)),
        out_shape=jax.ShapeDtypeStruct((B, S, H), x.dtype),
    )(x, pos)
